# baseline (speedup 1.0000x reference)
import functools

import jax
import jax.numpy as jnp
from jax.experimental import pallas as pl
from jax.experimental.pallas import tpu as pltpu

D = 768
DFF = 4 * D
E = 4
K = 2
ALT = 32
GH = D // 2
NEG_INF = float("-inf")


def _gelu_exact(x):
    return 0.5 * x * (1.0 + jax.lax.erf(x * (2.0 ** -0.5)))


def _router_body(nb, n_per_b, tok_ref, alt_ref, lnw_ref, lnb_ref,
                 gw1d_ref, gw1a_ref, gb1_ref, gw2_ref, gb2_ref,
                 tn32_ref, tnbf_ref, wcomb_ref, lb_ref, f_acc, p_acc):
    i = pl.program_id(0)
    x = tok_ref[...]
    mu = jnp.mean(x, axis=1, keepdims=True)
    xc = x - mu
    var = jnp.mean(xc * xc, axis=1, keepdims=True)
    tn = xc * jax.lax.rsqrt(var + 1e-5) * lnw_ref[...] + lnb_ref[...]
    tn32_ref[...] = tn
    tnbf_ref[...] = tn.astype(jnp.bfloat16)

    alt_c = jnp.dot(alt_ref[...], gw1a_ref[...],
                    preferred_element_type=jnp.float32)
    b = i // n_per_b
    sel = jax.lax.broadcasted_iota(jnp.int32, alt_c.shape, 0) == b
    ac = jnp.sum(jnp.where(sel, alt_c, 0.0), axis=0, keepdims=True)

    h_pre = jnp.dot(tn, gw1d_ref[...],
                    preferred_element_type=jnp.float32) + ac + gb1_ref[...]
    h = _gelu_exact(h_pre)
    logits = jnp.dot(h, gw2_ref[...],
                     preferred_element_type=jnp.float32) + gb2_ref[...]

    iota_e = jax.lax.broadcasted_iota(jnp.int32, logits.shape, 1)
    m0 = jnp.max(logits, axis=1, keepdims=True)
    e0 = jnp.min(jnp.where(logits == m0, iota_e, E), axis=1, keepdims=True)
    masked = jnp.where(iota_e == e0, NEG_INF, logits)
    m1 = jnp.max(masked, axis=1, keepdims=True)
    e1 = jnp.min(jnp.where(masked == m1, iota_e, E), axis=1, keepdims=True)

    z = jnp.exp(m1 - m0)
    g0 = 1.0 / (1.0 + z)
    g1 = z / (1.0 + z)

    p = jnp.exp(logits - m0)
    p = p / jnp.sum(p, axis=1, keepdims=True)

    wcomb_ref[...] = (jnp.where(iota_e == e0, g0, 0.0)
                      + jnp.where(iota_e == e1, g1, 0.0))

    f_part = jnp.sum((iota_e == e0).astype(jnp.float32), axis=0, keepdims=True)
    p_part = jnp.sum(p, axis=0, keepdims=True)

    @pl.when(i == 0)
    def _():
        f_acc[...] = f_part
        p_acc[...] = p_part

    @pl.when(i > 0)
    def _():
        f_acc[...] += f_part
        p_acc[...] += p_part

    @pl.when(i == nb - 1)
    def _():
        bn2 = float((nb * x.shape[0]) ** 2)
        lb_ref[...] = (E / bn2) * jnp.sum(f_acc[...] * p_acc[...],
                                          axis=1, keepdims=True)


def _run_router(tok2d, alt, lnw, lnb, gw1d, gw1a, gb1, gw2, gb2, n):
    bn = tok2d.shape[0]
    bt = 512
    nb = bn // bt
    n_per_b = n // bt
    body = functools.partial(_router_body, nb, n_per_b)
    return pl.pallas_call(
        body,
        grid=(nb,),
        in_specs=[
            pl.BlockSpec((bt, D), lambda i: (i, 0)),
            pl.BlockSpec(alt.shape, lambda i: (0, 0)),
            pl.BlockSpec((1, D), lambda i: (0, 0)),
            pl.BlockSpec((1, D), lambda i: (0, 0)),
            pl.BlockSpec((D, GH), lambda i: (0, 0)),
            pl.BlockSpec((ALT, GH), lambda i: (0, 0)),
            pl.BlockSpec((1, GH), lambda i: (0, 0)),
            pl.BlockSpec((GH, E), lambda i: (0, 0)),
            pl.BlockSpec((1, E), lambda i: (0, 0)),
        ],
        out_specs=[
            pl.BlockSpec((bt, D), lambda i: (i, 0)),
            pl.BlockSpec((bt, D), lambda i: (i, 0)),
            pl.BlockSpec((bt, E), lambda i: (i, 0)),
            pl.BlockSpec((1, 1), lambda i: (0, 0)),
        ],
        out_shape=[
            jax.ShapeDtypeStruct((bn, D), jnp.float32),
            jax.ShapeDtypeStruct((bn, D), jnp.bfloat16),
            jax.ShapeDtypeStruct((bn, E), jnp.float32),
            jax.ShapeDtypeStruct((1, 1), jnp.float32),
        ],
        scratch_shapes=[
            pltpu.VMEM((1, E), jnp.float32),
            pltpu.VMEM((1, E), jnp.float32),
        ],
    )(tok2d, alt, lnw, lnb, gw1d, gw1a, gb1, gw2, gb2)


DFFB = 768
NJ = DFF // DFFB
FBT = 2048
FNB = 2


def _dense_ffn_body(tnbf_ref, wcomb_ref, tok_ref, w1_ref, b1_ref,
                    w2_ref, b2_ref, out_ref):
    e = pl.program_id(0)
    j = pl.program_id(1)
    i = pl.program_id(2)
    rows = pl.ds(i * FBT, FBT)
    x = tnbf_ref[rows, :]
    w1 = w1_ref[0].astype(jnp.bfloat16)
    w2 = w2_ref[0].astype(jnp.bfloat16)
    h = jnp.dot(x, w1, preferred_element_type=jnp.float32)
    h = _gelu_exact(h + b1_ref[0])
    y = jnp.dot(h.astype(jnp.bfloat16), w2,
                preferred_element_type=jnp.float32)
    iota_e = jax.lax.broadcasted_iota(jnp.int32, (FBT, E), 1)
    w = jnp.sum(jnp.where(iota_e == e, wcomb_ref[rows, :], 0.0),
                axis=1, keepdims=True)

    @pl.when(j == 0)
    def _():
        y_b = y + b2_ref[0]

        @pl.when(e == 0)
        def _():
            out_ref[rows, :] = tok_ref[rows, :] + w * y_b

        @pl.when(e > 0)
        def _():
            out_ref[rows, :] += w * y_b

    @pl.when(j > 0)
    def _():
        out_ref[rows, :] += w * y


def _run_dense_ffn(tnbf, wcomb, tok2d, w1, b1, w2, b2):
    bn = tnbf.shape[0]
    return pl.pallas_call(
        _dense_ffn_body,
        grid=(E, NJ, FNB),
        in_specs=[
            pl.BlockSpec((bn, D), lambda e, j, i: (0, 0)),
            pl.BlockSpec((bn, E), lambda e, j, i: (0, 0)),
            pl.BlockSpec((bn, D), lambda e, j, i: (0, 0)),
            pl.BlockSpec((1, D, DFFB), lambda e, j, i: (e, 0, j)),
            pl.BlockSpec((1, 1, DFFB), lambda e, j, i: (e, 0, j)),
            pl.BlockSpec((1, DFFB, D), lambda e, j, i: (e, j, 0)),
            pl.BlockSpec((1, 1, D), lambda e, j, i: (e, 0, 0)),
        ],
        out_specs=pl.BlockSpec((bn, D), lambda e, j, i: (0, 0)),
        out_shape=jax.ShapeDtypeStruct((bn, D), jnp.float32),
    )(tnbf, wcomb, tok2d, w1, b1, w2, b2)


def kernel(tokens, alt_embedding, ln_w, ln_b, gate_w1, gate_b1, gate_w2,
           gate_b2, exp_w1, exp_b1, exp_w2, exp_b2):
    b, n, d = tokens.shape
    bn = b * n
    tok2d = tokens.reshape(bn, d)
    gw1d = gate_w1[:d]
    gw1a = gate_w1[d:]

    tn32, tnbf, wcomb, lb = _run_router(
        tok2d, alt_embedding, ln_w.reshape(1, d), ln_b.reshape(1, d),
        gw1d, gw1a, gate_b1.reshape(1, GH), gate_w2,
        gate_b2.reshape(1, E), n)

    out = _run_dense_ffn(
        tnbf, wcomb, tok2d,
        exp_w1, exp_b1.reshape(E, 1, DFF),
        exp_w2, exp_b2.reshape(E, 1, D))

    return (out.reshape(b, n, d), lb[0, 0])


# speedup vs baseline: 1.1060x; 1.1060x over previous
import functools

import jax
import jax.numpy as jnp
from jax.experimental import pallas as pl
from jax.experimental.pallas import tpu as pltpu

D = 768
DFF = 4 * D
E = 4
K = 2
ALT = 32
GH = D // 2
NEG_INF = float("-inf")


def _gelu_exact(x):
    return 0.5 * x * (1.0 + jax.lax.erf(x * (2.0 ** -0.5)))


def _router_body(nb, n_per_b, tok_ref, alt_ref, lnw_ref, lnb_ref,
                 gw1d_ref, gw1a_ref, gb1_ref, gw2_ref, gb2_ref,
                 tn32_ref, tnbf_ref, wcomb_ref, lb_ref, f_acc, p_acc):
    i = pl.program_id(0)
    x = tok_ref[...]
    mu = jnp.mean(x, axis=1, keepdims=True)
    xc = x - mu
    var = jnp.mean(xc * xc, axis=1, keepdims=True)
    tn = xc * jax.lax.rsqrt(var + 1e-5) * lnw_ref[...] + lnb_ref[...]
    tn32_ref[...] = tn
    tnbf_ref[...] = tn.astype(jnp.bfloat16)

    alt_c = jnp.dot(alt_ref[...], gw1a_ref[...],
                    preferred_element_type=jnp.float32)
    b = i // n_per_b
    sel = jax.lax.broadcasted_iota(jnp.int32, alt_c.shape, 0) == b
    ac = jnp.sum(jnp.where(sel, alt_c, 0.0), axis=0, keepdims=True)

    h_pre = jnp.dot(tn, gw1d_ref[...],
                    preferred_element_type=jnp.float32) + ac + gb1_ref[...]
    h = _gelu_exact(h_pre)
    logits = jnp.dot(h, gw2_ref[...],
                     preferred_element_type=jnp.float32) + gb2_ref[...]

    iota_e = jax.lax.broadcasted_iota(jnp.int32, logits.shape, 1)
    m0 = jnp.max(logits, axis=1, keepdims=True)
    e0 = jnp.min(jnp.where(logits == m0, iota_e, E), axis=1, keepdims=True)
    masked = jnp.where(iota_e == e0, NEG_INF, logits)
    m1 = jnp.max(masked, axis=1, keepdims=True)
    e1 = jnp.min(jnp.where(masked == m1, iota_e, E), axis=1, keepdims=True)

    z = jnp.exp(m1 - m0)
    g0 = 1.0 / (1.0 + z)
    g1 = z / (1.0 + z)

    p = jnp.exp(logits - m0)
    p = p / jnp.sum(p, axis=1, keepdims=True)

    wcomb_ref[...] = (jnp.where(iota_e == e0, g0, 0.0)
                      + jnp.where(iota_e == e1, g1, 0.0))

    f_part = jnp.sum((iota_e == e0).astype(jnp.float32), axis=0, keepdims=True)
    p_part = jnp.sum(p, axis=0, keepdims=True)

    @pl.when(i == 0)
    def _():
        f_acc[...] = f_part
        p_acc[...] = p_part

    @pl.when(i > 0)
    def _():
        f_acc[...] += f_part
        p_acc[...] += p_part

    @pl.when(i == nb - 1)
    def _():
        bn2 = float((nb * x.shape[0]) ** 2)
        lb_ref[...] = (E / bn2) * jnp.sum(f_acc[...] * p_acc[...],
                                          axis=1, keepdims=True)


def _run_router(tok2d, alt, lnw, lnb, gw1d, gw1a, gb1, gw2, gb2, n):
    bn = tok2d.shape[0]
    bt = 512
    nb = bn // bt
    n_per_b = n // bt
    body = functools.partial(_router_body, nb, n_per_b)
    return pl.pallas_call(
        body,
        grid=(nb,),
        in_specs=[
            pl.BlockSpec((bt, D), lambda i: (i, 0)),
            pl.BlockSpec(alt.shape, lambda i: (0, 0)),
            pl.BlockSpec((1, D), lambda i: (0, 0)),
            pl.BlockSpec((1, D), lambda i: (0, 0)),
            pl.BlockSpec((D, GH), lambda i: (0, 0)),
            pl.BlockSpec((ALT, GH), lambda i: (0, 0)),
            pl.BlockSpec((1, GH), lambda i: (0, 0)),
            pl.BlockSpec((GH, E), lambda i: (0, 0)),
            pl.BlockSpec((1, E), lambda i: (0, 0)),
        ],
        out_specs=[
            pl.BlockSpec((bt, D), lambda i: (i, 0)),
            pl.BlockSpec((bt, D), lambda i: (i, 0)),
            pl.BlockSpec((bt, E), lambda i: (i, 0)),
            pl.BlockSpec((1, 1), lambda i: (0, 0)),
        ],
        out_shape=[
            jax.ShapeDtypeStruct((bn, D), jnp.float32),
            jax.ShapeDtypeStruct((bn, D), jnp.bfloat16),
            jax.ShapeDtypeStruct((bn, E), jnp.float32),
            jax.ShapeDtypeStruct((1, 1), jnp.float32),
        ],
        scratch_shapes=[
            pltpu.VMEM((1, E), jnp.float32),
            pltpu.VMEM((1, E), jnp.float32),
        ],
    )(tok2d, alt, lnw, lnb, gw1d, gw1a, gb1, gw2, gb2)


DFFB = 1536
NJ = DFF // DFFB
FBT = 1024
FNB = 4


def _dense_ffn_body(tnbf_ref, wcomb_ref, tok_ref, w1_ref, b1_ref,
                    w2_ref, b2_ref, out_ref):
    e = pl.program_id(0)
    j = pl.program_id(1)
    i = pl.program_id(2)
    rows = pl.ds(i * FBT, FBT)
    x = tnbf_ref[rows, :]
    w1 = w1_ref[0].astype(jnp.bfloat16)
    w2 = w2_ref[0].astype(jnp.bfloat16)
    h = jnp.dot(x, w1, preferred_element_type=jnp.float32)
    h = _gelu_exact(h + b1_ref[0])
    y = jnp.dot(h.astype(jnp.bfloat16), w2,
                preferred_element_type=jnp.float32)
    iota_e = jax.lax.broadcasted_iota(jnp.int32, (FBT, E), 1)
    w = jnp.sum(jnp.where(iota_e == e, wcomb_ref[rows, :], 0.0),
                axis=1, keepdims=True)

    @pl.when(j == 0)
    def _():
        y_b = y + b2_ref[0]

        @pl.when(e == 0)
        def _():
            out_ref[rows, :] = tok_ref[rows, :] + w * y_b

        @pl.when(e > 0)
        def _():
            out_ref[rows, :] += w * y_b

    @pl.when(j > 0)
    def _():
        out_ref[rows, :] += w * y


def _run_dense_ffn(tnbf, wcomb, tok2d, w1, b1, w2, b2):
    bn = tnbf.shape[0]
    return pl.pallas_call(
        _dense_ffn_body,
        grid=(E, NJ, FNB),
        in_specs=[
            pl.BlockSpec((bn, D), lambda e, j, i: (0, 0)),
            pl.BlockSpec((bn, E), lambda e, j, i: (0, 0)),
            pl.BlockSpec((bn, D), lambda e, j, i: (0, 0)),
            pl.BlockSpec((1, D, DFFB), lambda e, j, i: (e, 0, j)),
            pl.BlockSpec((1, 1, DFFB), lambda e, j, i: (e, 0, j)),
            pl.BlockSpec((1, DFFB, D), lambda e, j, i: (e, j, 0)),
            pl.BlockSpec((1, 1, D), lambda e, j, i: (e, 0, 0)),
        ],
        out_specs=pl.BlockSpec((bn, D), lambda e, j, i: (0, 0)),
        out_shape=jax.ShapeDtypeStruct((bn, D), jnp.float32),
    )(tnbf, wcomb, tok2d, w1, b1, w2, b2)


def kernel(tokens, alt_embedding, ln_w, ln_b, gate_w1, gate_b1, gate_w2,
           gate_b2, exp_w1, exp_b1, exp_w2, exp_b2):
    b, n, d = tokens.shape
    bn = b * n
    tok2d = tokens.reshape(bn, d)
    gw1d = gate_w1[:d]
    gw1a = gate_w1[d:]

    tn32, tnbf, wcomb, lb = _run_router(
        tok2d, alt_embedding, ln_w.reshape(1, d), ln_b.reshape(1, d),
        gw1d, gw1a, gate_b1.reshape(1, GH), gate_w2,
        gate_b2.reshape(1, E), n)

    out = _run_dense_ffn(
        tnbf, wcomb, tok2d,
        exp_w1, exp_b1.reshape(E, 1, DFF),
        exp_w2, exp_b2.reshape(E, 1, D))

    return (out.reshape(b, n, d), lb[0, 0])
